# baseline (speedup 1.0000x reference)
import jax
import jax.numpy as jnp
from jax import lax
from jax.experimental import pallas as pl
from jax.experimental.pallas import tpu as pltpu
from jax.experimental.pallas import tpu_sc as plsc

N = 10000
E = 320000
C = 128
NP = 10112
NTILES = 32
ROWS_PER_TILE = 80
NROWS = NTILES * ROWS_PER_TILE
EPAD = NROWS * 128
SLAB = NP // 16
W0_SC2 = 128
W1_SC2 = 32
F32 = jnp.float32


def _ka1_body(x_ref, wt_ref, asv_ref, adv_ref, xs_ref, asrc_ref, adst_ref):
    xb = jnp.dot(x_ref[...], wt_ref[...], preferred_element_type=F32)
    xs_ref[...] = xb
    asrc_ref[...] = jnp.sum(xb * asv_ref[...], axis=1, keepdims=True)
    adst_ref[...] = jnp.sum(xb * adv_ref[...], axis=1, keepdims=True)


def _ka1(x, wt, asv, adv):
    bn = 1000
    return pl.pallas_call(
        _ka1_body,
        grid=(N // bn,),
        in_specs=[
            pl.BlockSpec((bn, C), lambda i: (i, 0)),
            pl.BlockSpec((C, C), lambda i: (0, 0)),
            pl.BlockSpec((1, C), lambda i: (0, 0)),
            pl.BlockSpec((1, C), lambda i: (0, 0)),
        ],
        out_specs=[
            pl.BlockSpec((bn, C), lambda i: (i, 0)),
            pl.BlockSpec((bn, 1), lambda i: (i, 0)),
            pl.BlockSpec((bn, 1), lambda i: (i, 0)),
        ],
        out_shape=[
            jax.ShapeDtypeStruct((N, C), F32),
            jax.ShapeDtypeStruct((N, 1), F32),
            jax.ShapeDtypeStruct((N, 1), F32),
        ],
    )(x, wt, asv, adv)


def _ksc1_body(src2d, dst2d, ea0, ea1, ea2, asrc_h, adst_h, v16_h, znp_h,
               e2d, den_part, deg_part, sae_part,
               asrc_t, adst_t, den_t, deg_t, sae_t,
               src_sb, dst_sb, ea0_sb, ea1_sb, ea2_sb, e_sb, vbuf):
    cid = lax.axis_index("c")
    sid = lax.axis_index("s")
    wid = sid * 2 + cid

    pltpu.sync_copy(asrc_h, asrc_t)
    pltpu.sync_copy(adst_h, adst_t)
    pltpu.sync_copy(znp_h, den_t)
    pltpu.sync_copy(znp_h, deg_t)
    pltpu.sync_copy(znp_h, sae_t)
    pltpu.sync_copy(v16_h, vbuf)
    idx0 = jnp.zeros((16,), jnp.int32)
    v0b = plsc.load_gather(vbuf, [idx0 + 1])
    v1b = plsc.load_gather(vbuf, [idx0 + 2])
    v2b = plsc.load_gather(vbuf, [idx0 + 3])
    ones16 = jnp.full((16,), 1.0, F32)

    ns = 8
    for s5 in range(ROWS_PER_TILE // ns):
        r0 = wid * ROWS_PER_TILE + s5 * ns
        pltpu.sync_copy(src2d.at[pl.ds(r0, ns)], src_sb)
        pltpu.sync_copy(dst2d.at[pl.ds(r0, ns)], dst_sb)
        pltpu.sync_copy(ea0.at[pl.ds(r0, ns)], ea0_sb)
        pltpu.sync_copy(ea1.at[pl.ds(r0, ns)], ea1_sb)
        pltpu.sync_copy(ea2.at[pl.ds(r0, ns)], ea2_sb)

        def row_body(j, carry):
            for k in range(8):
                sl = pl.ds(16 * k, 16)
                sv = src_sb[j, sl]
                dv = dst_sb[j, sl]
                ae = ea0_sb[j, sl] * v0b + ea1_sb[j, sl] * v1b + ea2_sb[j, sl] * v2b
                al = (plsc.load_gather(asrc_t, [sv])
                      + plsc.load_gather(adst_t, [dv]) + ae)
                al = jnp.where(al >= 0.0, al, al * 0.2)
                ev = jnp.exp(al)
                e_sb[j, sl] = ev
                plsc.addupdate_scatter(den_t, [dv], ev)
                plsc.addupdate_scatter(deg_t, [dv], ones16)
                plsc.addupdate_scatter(sae_t, [dv], ae)
            return carry

        lax.fori_loop(0, ns, row_body, 0)
        pltpu.sync_copy(e_sb, e2d.at[pl.ds(r0, ns)])

    pltpu.sync_copy(den_t, den_part.at[pl.ds(wid * NP, NP)])
    pltpu.sync_copy(deg_t, deg_part.at[pl.ds(wid * NP, NP)])
    pltpu.sync_copy(sae_t, sae_part.at[pl.ds(wid * NP, NP)])


def _ksc1(src2d, dst2d, ea0, ea1, ea2, asrc_p, adst_p, v16, znp):
    mesh = plsc.VectorSubcoreMesh(core_axis_name="c", subcore_axis_name="s",
                                  num_cores=2, num_subcores=16)
    f = pl.kernel(
        _ksc1_body,
        out_type=(
            jax.ShapeDtypeStruct((NROWS, 128), F32),
            jax.ShapeDtypeStruct((NTILES * NP,), F32),
            jax.ShapeDtypeStruct((NTILES * NP,), F32),
            jax.ShapeDtypeStruct((NTILES * NP,), F32),
        ),
        mesh=mesh,
        compiler_params=pltpu.CompilerParams(needs_layout_passes=False),
        scratch_types=[
            pltpu.VMEM((NP,), F32),
            pltpu.VMEM((NP,), F32),
            pltpu.VMEM((NP,), F32),
            pltpu.VMEM((NP,), F32),
            pltpu.VMEM((NP,), F32),
            pltpu.VMEM((8, 128), jnp.int32),
            pltpu.VMEM((8, 128), jnp.int32),
            pltpu.VMEM((8, 128), F32),
            pltpu.VMEM((8, 128), F32),
            pltpu.VMEM((8, 128), F32),
            pltpu.VMEM((8, 128), F32),
            pltpu.VMEM((16,), F32),
        ],
    )
    return f(src2d, dst2d, ea0, ea1, ea2, asrc_p, adst_p, v16, znp)


def _ksc2_body(src2d, dst2d, e2d, xs_h, zrow_h,
               out_part,
               src_sb, dst_sb, e_sb, buf0, buf1, acc_sh, sem0, sem1):
    cid = lax.axis_index("c")
    sid = lax.axis_index("s")

    pltpu.sync_copy(zrow_h, buf0)
    slab = sid * SLAB
    for i in range(4):
        pltpu.sync_copy(buf0, acc_sh.at[pl.ds(slab + i * 128, 128)])
    pltpu.sync_copy(buf0.at[pl.ds(0, SLAB - 512)],
                    acc_sh.at[pl.ds(slab + 512, SLAB - 512)])
    plsc.subcore_barrier()

    ns = 8
    base = jnp.where(cid == 0, sid * W0_SC2, 16 * W0_SC2 + sid * W1_SC2)
    nsup0 = W0_SC2 // ns
    nsup1 = W1_SC2 // ns

    def scale_scatter(j, buf):
        @plsc.parallel_loop(0, 128, unroll=4)
        def scale_body(r):
            sb = plsc.load_gather(
                e_sb, [jnp.full((16,), j, jnp.int32),
                       jnp.full((16,), r, jnp.int32)])
            for c in range(8):
                slc = pl.ds(16 * c, 16)
                buf[r, slc] = buf[r, slc] * sb

        pltpu.sync_copy(buf, acc_sh.at[dst_sb.at[j]], add=True)

    for s5 in range(max(nsup0, nsup1)):
        if s5 >= min(nsup0, nsup1):
            other = 1 if nsup1 > nsup0 else 0
            guard = lambda: pl.when(cid == other)
        else:
            guard = None
        r0 = base + s5 * ns

        def super_chunk(r0=r0):
            pltpu.sync_copy(src2d.at[pl.ds(r0, ns)], src_sb)
            pltpu.sync_copy(dst2d.at[pl.ds(r0, ns)], dst_sb)
            pltpu.sync_copy(e2d.at[pl.ds(r0, ns)], e_sb)
            pltpu.async_copy(xs_h.at[src_sb.at[0]], buf0, sem0)

            def pair_body(t, carry):
                j0 = 2 * t
                j1 = j0 + 1
                pltpu.make_async_copy(xs_h.at[src_sb.at[j0]], buf0,
                                      sem0).wait()
                pltpu.async_copy(xs_h.at[src_sb.at[j1]], buf1, sem1)
                scale_scatter(j0, buf0)
                pltpu.make_async_copy(xs_h.at[src_sb.at[j1]], buf1,
                                      sem1).wait()

                @pl.when(t < ns // 2 - 1)
                def _():
                    pltpu.async_copy(xs_h.at[src_sb.at[j0 + 2]], buf0, sem0)
                scale_scatter(j1, buf1)
                return carry

            lax.fori_loop(0, ns // 2, pair_body, 0)

        if guard is None:
            super_chunk()
        else:
            guard()(super_chunk)

    plsc.subcore_barrier()
    pltpu.sync_copy(acc_sh.at[pl.ds(slab, SLAB)],
                    out_part.at[cid, pl.ds(slab, SLAB)])


def _ksc2(src2d, dst2d, e2d, xs, zrow):
    mesh = plsc.VectorSubcoreMesh(core_axis_name="c", subcore_axis_name="s",
                                  num_cores=2, num_subcores=16)
    f = pl.kernel(
        _ksc2_body,
        out_type=jax.ShapeDtypeStruct((2, NP, C), F32),
        mesh=mesh,
        compiler_params=pltpu.CompilerParams(needs_layout_passes=False),
        scratch_types=[
            pltpu.VMEM((8, 128), jnp.int32),
            pltpu.VMEM((8, 128), jnp.int32),
            pltpu.VMEM((8, 128), F32),
            pltpu.VMEM((128, C), F32),
            pltpu.VMEM((128, C), F32),
            pltpu.VMEM_SHARED((NP, C), F32),
            pltpu.SemaphoreType.DMA,
            pltpu.SemaphoreType.DMA,
        ],
    )
    return f(src2d, dst2d, e2d, xs, zrow)


def _k4a_body(op0_ref, op1_ref, xs_ref, denp_ref, degp_ref, saep_ref,
              as_ref, ad_ref, h_ref, s_ref, q_ref):
    den = jnp.sum(denp_ref[...], axis=1)
    deg = jnp.sum(degp_ref[...], axis=1)
    sae = jnp.sum(saep_ref[...], axis=1)
    al = as_ref[...][:, 0] + ad_ref[...][:, 0] + sae / jnp.maximum(deg, 1.0)
    al = jnp.where(al >= 0.0, al, al * 0.2)
    el = jnp.exp(al)
    rden = 1.0 / (den + el + 1e-16)
    h = (op0_ref[...] + op1_ref[...] + el[:, None] * xs_ref[...]) * rden[:, None]
    h_ref[...] = h

    @pl.when(pl.program_id(0) == 0)
    def _():
        s_ref[...] = jnp.zeros_like(s_ref)
        q_ref[...] = jnp.zeros_like(q_ref)

    s_ref[...] += jnp.sum(h, axis=0, keepdims=True)
    q_ref[...] += jnp.sum(h * h, axis=0, keepdims=True)


def _k4a(op0, op1, xs, denp, degp, saep, asrc, adst):
    bn = 1000
    return pl.pallas_call(
        _k4a_body,
        grid=(N // bn,),
        in_specs=[
            pl.BlockSpec((bn, C), lambda i: (i, 0)),
            pl.BlockSpec((bn, C), lambda i: (i, 0)),
            pl.BlockSpec((bn, C), lambda i: (i, 0)),
            pl.BlockSpec((bn, NTILES), lambda i: (i, 0)),
            pl.BlockSpec((bn, NTILES), lambda i: (i, 0)),
            pl.BlockSpec((bn, NTILES), lambda i: (i, 0)),
            pl.BlockSpec((bn, 1), lambda i: (i, 0)),
            pl.BlockSpec((bn, 1), lambda i: (i, 0)),
        ],
        out_specs=[
            pl.BlockSpec((bn, C), lambda i: (i, 0)),
            pl.BlockSpec((1, C), lambda i: (0, 0)),
            pl.BlockSpec((1, C), lambda i: (0, 0)),
        ],
        out_shape=[
            jax.ShapeDtypeStruct((N, C), F32),
            jax.ShapeDtypeStruct((1, C), F32),
            jax.ShapeDtypeStruct((1, C), F32),
        ],
    )(op0, op1, xs, denp, degp, saep, asrc, adst)


def _k4b_body(h_ref, sc0_ref, sh0_ref, lwt_ref, z_ref, s_ref, q_ref):
    y = jnp.maximum(h_ref[...] * sc0_ref[...] + sh0_ref[...], 0.0)
    z = jnp.dot(y, lwt_ref[...], preferred_element_type=F32)
    z_ref[...] = z

    @pl.when(pl.program_id(0) == 0)
    def _():
        s_ref[...] = jnp.zeros_like(s_ref)
        q_ref[...] = jnp.zeros_like(q_ref)

    s_ref[...] += jnp.sum(z, axis=0, keepdims=True)
    q_ref[...] += jnp.sum(z * z, axis=0, keepdims=True)


def _k4b(h, sc0, sh0, lwt):
    bn = 1000
    return pl.pallas_call(
        _k4b_body,
        grid=(N // bn,),
        in_specs=[
            pl.BlockSpec((bn, C), lambda i: (i, 0)),
            pl.BlockSpec((1, C), lambda i: (0, 0)),
            pl.BlockSpec((1, C), lambda i: (0, 0)),
            pl.BlockSpec((C, C), lambda i: (0, 0)),
        ],
        out_specs=[
            pl.BlockSpec((bn, C), lambda i: (i, 0)),
            pl.BlockSpec((1, C), lambda i: (0, 0)),
            pl.BlockSpec((1, C), lambda i: (0, 0)),
        ],
        out_shape=[
            jax.ShapeDtypeStruct((N, C), F32),
            jax.ShapeDtypeStruct((1, C), F32),
            jax.ShapeDtypeStruct((1, C), F32),
        ],
    )(h, sc0, sh0, lwt)


def _k4c_body(z_ref, sc1_ref, sh1_ref, o_ref):
    o_ref[...] = jnp.maximum(z_ref[...] * sc1_ref[...] + sh1_ref[...], 0.0)


def _k4c(z, sc1, sh1):
    bn = 1000
    return pl.pallas_call(
        _k4c_body,
        grid=(N // bn,),
        in_specs=[
            pl.BlockSpec((bn, C), lambda i: (i, 0)),
            pl.BlockSpec((1, C), lambda i: (0, 0)),
            pl.BlockSpec((1, C), lambda i: (0, 0)),
        ],
        out_specs=pl.BlockSpec((bn, C), lambda i: (i, 0)),
        out_shape=jax.ShapeDtypeStruct((N, C), F32),
    )(z, sc1, sh1)


def kernel(x, adj, edge_attr, W, att_src, att_dst, W_edge, att_edge, bias,
           bn0_gamma, bn0_beta, lin_W, lin_b, bn1_gamma, bn1_beta):
    del bias, lin_b
    src, dst = adj[0], adj[1]

    v3 = att_edge.reshape(-1) @ W_edge
    v16 = jnp.zeros((16,), F32).at[1:4].set(v3)
    asv = att_src.reshape(1, C)
    adv = att_dst.reshape(1, C)

    pad = EPAD - E
    srcp = jnp.concatenate([src, jnp.zeros((pad,), src.dtype)]).reshape(NROWS, 128)
    dstp = jnp.concatenate([dst, jnp.full((pad,), N, dst.dtype)]).reshape(NROWS, 128)
    eap = jnp.pad(edge_attr, ((0, pad), (0, 0)))
    ea0 = eap[:, 0].reshape(NROWS, 128)
    ea1 = eap[:, 1].reshape(NROWS, 128)
    ea2 = eap[:, 2].reshape(NROWS, 128)

    xs, asrc, adst = _ka1(x, W.T, asv, adv)

    asrc_p = jnp.pad(asrc[:, 0], (0, NP - N))
    adst_p = jnp.pad(adst[:, 0], (0, NP - N))
    znp = jnp.zeros((NP,), F32)
    zrow = jnp.zeros((128, C), F32)

    e2d, den_part, deg_part, sae_part = _ksc1(
        srcp, dstp, ea0, ea1, ea2, asrc_p, adst_p, v16, znp)

    out_part = _ksc2(srcp, dstp, e2d, xs, zrow)

    denp = den_part.reshape(NTILES, NP)[:, :N].T
    degp = deg_part.reshape(NTILES, NP)[:, :N].T
    saep = sae_part.reshape(NTILES, NP)[:, :N].T
    h, s0, q0 = _k4a(out_part[0, :N, :], out_part[1, :N, :], xs,
                     denp, degp, saep, asrc, adst)
    m0 = s0 / N
    v0 = q0 / N - m0 * m0
    sc0 = bn0_gamma.reshape(1, C) / jnp.sqrt(v0 + 1e-5)
    sh0 = bn0_beta.reshape(1, C) - m0 * sc0

    z, s1, q1 = _k4b(h, sc0, sh0, lin_W.T)
    m1 = s1 / N
    v1 = q1 / N - m1 * m1
    sc1 = bn1_gamma.reshape(1, C) / jnp.sqrt(v1 + 1e-5)
    sh1 = bn1_beta.reshape(1, C) - m1 * sc1

    return _k4c(z, sc1, sh1)


# speedup vs baseline: 1.0420x; 1.0420x over previous
import jax
import jax.numpy as jnp
from jax import lax
from jax.experimental import pallas as pl
from jax.experimental.pallas import tpu as pltpu
from jax.experimental.pallas import tpu_sc as plsc

N = 10000
E = 320000
C = 128
NP = 10112
NTILES = 32
ROWS_PER_TILE = 80
NROWS = NTILES * ROWS_PER_TILE
EPAD = NROWS * 128
SLAB = NP // 16
W0_SC2 = 152
W1_SC2 = 8
F32 = jnp.float32


def _ka1_body(x_ref, wt_ref, asv_ref, adv_ref, xs_ref, asrc_ref, adst_ref):
    xb = jnp.dot(x_ref[...], wt_ref[...], preferred_element_type=F32)
    xs_ref[...] = xb
    asrc_ref[...] = jnp.sum(xb * asv_ref[...], axis=1, keepdims=True)
    adst_ref[...] = jnp.sum(xb * adv_ref[...], axis=1, keepdims=True)


def _ka1(x, wt, asv, adv):
    bn = 1000
    return pl.pallas_call(
        _ka1_body,
        grid=(N // bn,),
        in_specs=[
            pl.BlockSpec((bn, C), lambda i: (i, 0)),
            pl.BlockSpec((C, C), lambda i: (0, 0)),
            pl.BlockSpec((1, C), lambda i: (0, 0)),
            pl.BlockSpec((1, C), lambda i: (0, 0)),
        ],
        out_specs=[
            pl.BlockSpec((bn, C), lambda i: (i, 0)),
            pl.BlockSpec((bn, 1), lambda i: (i, 0)),
            pl.BlockSpec((bn, 1), lambda i: (i, 0)),
        ],
        out_shape=[
            jax.ShapeDtypeStruct((N, C), F32),
            jax.ShapeDtypeStruct((N, 1), F32),
            jax.ShapeDtypeStruct((N, 1), F32),
        ],
    )(x, wt, asv, adv)


def _ksc1_body(src2d, dst2d, ea0, ea1, ea2, asrc_h, adst_h, v16_h, znp_h,
               e2d, den_part, deg_part, sae_part,
               asrc_t, adst_t, den_t, deg_t, sae_t,
               src_sb, dst_sb, ea0_sb, ea1_sb, ea2_sb, e_sb, vbuf):
    cid = lax.axis_index("c")
    sid = lax.axis_index("s")
    wid = sid * 2 + cid

    pltpu.sync_copy(asrc_h, asrc_t)
    pltpu.sync_copy(adst_h, adst_t)
    pltpu.sync_copy(znp_h, den_t)
    pltpu.sync_copy(znp_h, deg_t)
    pltpu.sync_copy(znp_h, sae_t)
    pltpu.sync_copy(v16_h, vbuf)
    idx0 = jnp.zeros((16,), jnp.int32)
    v0b = plsc.load_gather(vbuf, [idx0 + 1])
    v1b = plsc.load_gather(vbuf, [idx0 + 2])
    v2b = plsc.load_gather(vbuf, [idx0 + 3])
    ones16 = jnp.full((16,), 1.0, F32)

    ns = 8
    for s5 in range(ROWS_PER_TILE // ns):
        r0 = wid * ROWS_PER_TILE + s5 * ns
        pltpu.sync_copy(src2d.at[pl.ds(r0, ns)], src_sb)
        pltpu.sync_copy(dst2d.at[pl.ds(r0, ns)], dst_sb)
        pltpu.sync_copy(ea0.at[pl.ds(r0, ns)], ea0_sb)
        pltpu.sync_copy(ea1.at[pl.ds(r0, ns)], ea1_sb)
        pltpu.sync_copy(ea2.at[pl.ds(r0, ns)], ea2_sb)

        def row_body(j, carry):
            for k in range(8):
                sl = pl.ds(16 * k, 16)
                sv = src_sb[j, sl]
                dv = dst_sb[j, sl]
                ae = ea0_sb[j, sl] * v0b + ea1_sb[j, sl] * v1b + ea2_sb[j, sl] * v2b
                al = (plsc.load_gather(asrc_t, [sv])
                      + plsc.load_gather(adst_t, [dv]) + ae)
                al = jnp.where(al >= 0.0, al, al * 0.2)
                ev = jnp.exp(al)
                e_sb[j, sl] = ev
                plsc.addupdate_scatter(den_t, [dv], ev)
                plsc.addupdate_scatter(deg_t, [dv], ones16)
                plsc.addupdate_scatter(sae_t, [dv], ae)
            return carry

        lax.fori_loop(0, ns, row_body, 0)
        pltpu.sync_copy(e_sb, e2d.at[pl.ds(r0, ns)])

    pltpu.sync_copy(den_t, den_part.at[pl.ds(wid * NP, NP)])
    pltpu.sync_copy(deg_t, deg_part.at[pl.ds(wid * NP, NP)])
    pltpu.sync_copy(sae_t, sae_part.at[pl.ds(wid * NP, NP)])


def _ksc1(src2d, dst2d, ea0, ea1, ea2, asrc_p, adst_p, v16, znp):
    mesh = plsc.VectorSubcoreMesh(core_axis_name="c", subcore_axis_name="s",
                                  num_cores=2, num_subcores=16)
    f = pl.kernel(
        _ksc1_body,
        out_type=(
            jax.ShapeDtypeStruct((NROWS, 128), F32),
            jax.ShapeDtypeStruct((NTILES * NP,), F32),
            jax.ShapeDtypeStruct((NTILES * NP,), F32),
            jax.ShapeDtypeStruct((NTILES * NP,), F32),
        ),
        mesh=mesh,
        compiler_params=pltpu.CompilerParams(needs_layout_passes=False),
        scratch_types=[
            pltpu.VMEM((NP,), F32),
            pltpu.VMEM((NP,), F32),
            pltpu.VMEM((NP,), F32),
            pltpu.VMEM((NP,), F32),
            pltpu.VMEM((NP,), F32),
            pltpu.VMEM((8, 128), jnp.int32),
            pltpu.VMEM((8, 128), jnp.int32),
            pltpu.VMEM((8, 128), F32),
            pltpu.VMEM((8, 128), F32),
            pltpu.VMEM((8, 128), F32),
            pltpu.VMEM((8, 128), F32),
            pltpu.VMEM((16,), F32),
        ],
    )
    return f(src2d, dst2d, ea0, ea1, ea2, asrc_p, adst_p, v16, znp)


def _ksc2_body(src2d, dst2d, e2d, xs_h, zrow_h,
               out_part,
               src_sb, dst_sb, e_sb, buf0, buf1, acc_sh, sem0, sem1):
    cid = lax.axis_index("c")
    sid = lax.axis_index("s")

    pltpu.sync_copy(zrow_h, buf0)
    slab = sid * SLAB
    for i in range(4):
        pltpu.sync_copy(buf0, acc_sh.at[pl.ds(slab + i * 128, 128)])
    pltpu.sync_copy(buf0.at[pl.ds(0, SLAB - 512)],
                    acc_sh.at[pl.ds(slab + 512, SLAB - 512)])
    plsc.subcore_barrier()

    ns = 8
    base = jnp.where(cid == 0, sid * W0_SC2, 16 * W0_SC2 + sid * W1_SC2)
    nsup0 = W0_SC2 // ns
    nsup1 = W1_SC2 // ns

    def scale_scatter(j, buf):
        @plsc.parallel_loop(0, 128, unroll=4)
        def scale_body(r):
            sb = plsc.load_gather(
                e_sb, [jnp.full((16,), j, jnp.int32),
                       jnp.full((16,), r, jnp.int32)])
            for c in range(8):
                slc = pl.ds(16 * c, 16)
                buf[r, slc] = buf[r, slc] * sb

        pltpu.sync_copy(buf, acc_sh.at[dst_sb.at[j]], add=True)

    for s5 in range(max(nsup0, nsup1)):
        if s5 >= min(nsup0, nsup1):
            other = 1 if nsup1 > nsup0 else 0
            guard = lambda: pl.when(cid == other)
        else:
            guard = None
        r0 = base + s5 * ns

        def super_chunk(r0=r0):
            pltpu.sync_copy(src2d.at[pl.ds(r0, ns)], src_sb)
            pltpu.sync_copy(dst2d.at[pl.ds(r0, ns)], dst_sb)
            pltpu.sync_copy(e2d.at[pl.ds(r0, ns)], e_sb)
            pltpu.async_copy(xs_h.at[src_sb.at[0]], buf0, sem0)

            def pair_body(t, carry):
                j0 = 2 * t
                j1 = j0 + 1
                pltpu.make_async_copy(xs_h.at[src_sb.at[j0]], buf0,
                                      sem0).wait()
                pltpu.async_copy(xs_h.at[src_sb.at[j1]], buf1, sem1)
                scale_scatter(j0, buf0)
                pltpu.make_async_copy(xs_h.at[src_sb.at[j1]], buf1,
                                      sem1).wait()

                @pl.when(t < ns // 2 - 1)
                def _():
                    pltpu.async_copy(xs_h.at[src_sb.at[j0 + 2]], buf0, sem0)
                scale_scatter(j1, buf1)
                return carry

            lax.fori_loop(0, ns // 2, pair_body, 0)

        if guard is None:
            super_chunk()
        else:
            guard()(super_chunk)

    plsc.subcore_barrier()
    pltpu.sync_copy(acc_sh.at[pl.ds(slab, SLAB)],
                    out_part.at[cid, pl.ds(slab, SLAB)])


def _ksc2(src2d, dst2d, e2d, xs, zrow):
    mesh = plsc.VectorSubcoreMesh(core_axis_name="c", subcore_axis_name="s",
                                  num_cores=2, num_subcores=16)
    f = pl.kernel(
        _ksc2_body,
        out_type=jax.ShapeDtypeStruct((2, NP, C), F32),
        mesh=mesh,
        compiler_params=pltpu.CompilerParams(needs_layout_passes=False),
        scratch_types=[
            pltpu.VMEM((8, 128), jnp.int32),
            pltpu.VMEM((8, 128), jnp.int32),
            pltpu.VMEM((8, 128), F32),
            pltpu.VMEM((128, C), F32),
            pltpu.VMEM((128, C), F32),
            pltpu.VMEM_SHARED((NP, C), F32),
            pltpu.SemaphoreType.DMA,
            pltpu.SemaphoreType.DMA,
        ],
    )
    return f(src2d, dst2d, e2d, xs, zrow)


def _k4a_body(op0_ref, op1_ref, xs_ref, denp_ref, degp_ref, saep_ref,
              as_ref, ad_ref, h_ref, s_ref, q_ref):
    den = jnp.sum(denp_ref[...], axis=1)
    deg = jnp.sum(degp_ref[...], axis=1)
    sae = jnp.sum(saep_ref[...], axis=1)
    al = as_ref[...][:, 0] + ad_ref[...][:, 0] + sae / jnp.maximum(deg, 1.0)
    al = jnp.where(al >= 0.0, al, al * 0.2)
    el = jnp.exp(al)
    rden = 1.0 / (den + el + 1e-16)
    h = (op0_ref[...] + op1_ref[...] + el[:, None] * xs_ref[...]) * rden[:, None]
    h_ref[...] = h

    @pl.when(pl.program_id(0) == 0)
    def _():
        s_ref[...] = jnp.zeros_like(s_ref)
        q_ref[...] = jnp.zeros_like(q_ref)

    s_ref[...] += jnp.sum(h, axis=0, keepdims=True)
    q_ref[...] += jnp.sum(h * h, axis=0, keepdims=True)


def _k4a(op0, op1, xs, denp, degp, saep, asrc, adst):
    bn = 1000
    return pl.pallas_call(
        _k4a_body,
        grid=(N // bn,),
        in_specs=[
            pl.BlockSpec((bn, C), lambda i: (i, 0)),
            pl.BlockSpec((bn, C), lambda i: (i, 0)),
            pl.BlockSpec((bn, C), lambda i: (i, 0)),
            pl.BlockSpec((bn, NTILES), lambda i: (i, 0)),
            pl.BlockSpec((bn, NTILES), lambda i: (i, 0)),
            pl.BlockSpec((bn, NTILES), lambda i: (i, 0)),
            pl.BlockSpec((bn, 1), lambda i: (i, 0)),
            pl.BlockSpec((bn, 1), lambda i: (i, 0)),
        ],
        out_specs=[
            pl.BlockSpec((bn, C), lambda i: (i, 0)),
            pl.BlockSpec((1, C), lambda i: (0, 0)),
            pl.BlockSpec((1, C), lambda i: (0, 0)),
        ],
        out_shape=[
            jax.ShapeDtypeStruct((N, C), F32),
            jax.ShapeDtypeStruct((1, C), F32),
            jax.ShapeDtypeStruct((1, C), F32),
        ],
    )(op0, op1, xs, denp, degp, saep, asrc, adst)


def _k4b_body(h_ref, sc0_ref, sh0_ref, lwt_ref, z_ref, s_ref, q_ref):
    y = jnp.maximum(h_ref[...] * sc0_ref[...] + sh0_ref[...], 0.0)
    z = jnp.dot(y, lwt_ref[...], preferred_element_type=F32)
    z_ref[...] = z

    @pl.when(pl.program_id(0) == 0)
    def _():
        s_ref[...] = jnp.zeros_like(s_ref)
        q_ref[...] = jnp.zeros_like(q_ref)

    s_ref[...] += jnp.sum(z, axis=0, keepdims=True)
    q_ref[...] += jnp.sum(z * z, axis=0, keepdims=True)


def _k4b(h, sc0, sh0, lwt):
    bn = 1000
    return pl.pallas_call(
        _k4b_body,
        grid=(N // bn,),
        in_specs=[
            pl.BlockSpec((bn, C), lambda i: (i, 0)),
            pl.BlockSpec((1, C), lambda i: (0, 0)),
            pl.BlockSpec((1, C), lambda i: (0, 0)),
            pl.BlockSpec((C, C), lambda i: (0, 0)),
        ],
        out_specs=[
            pl.BlockSpec((bn, C), lambda i: (i, 0)),
            pl.BlockSpec((1, C), lambda i: (0, 0)),
            pl.BlockSpec((1, C), lambda i: (0, 0)),
        ],
        out_shape=[
            jax.ShapeDtypeStruct((N, C), F32),
            jax.ShapeDtypeStruct((1, C), F32),
            jax.ShapeDtypeStruct((1, C), F32),
        ],
    )(h, sc0, sh0, lwt)


def _k4c_body(z_ref, sc1_ref, sh1_ref, o_ref):
    o_ref[...] = jnp.maximum(z_ref[...] * sc1_ref[...] + sh1_ref[...], 0.0)


def _k4c(z, sc1, sh1):
    bn = 1000
    return pl.pallas_call(
        _k4c_body,
        grid=(N // bn,),
        in_specs=[
            pl.BlockSpec((bn, C), lambda i: (i, 0)),
            pl.BlockSpec((1, C), lambda i: (0, 0)),
            pl.BlockSpec((1, C), lambda i: (0, 0)),
        ],
        out_specs=pl.BlockSpec((bn, C), lambda i: (i, 0)),
        out_shape=jax.ShapeDtypeStruct((N, C), F32),
    )(z, sc1, sh1)


def kernel(x, adj, edge_attr, W, att_src, att_dst, W_edge, att_edge, bias,
           bn0_gamma, bn0_beta, lin_W, lin_b, bn1_gamma, bn1_beta):
    del bias, lin_b
    src, dst = adj[0], adj[1]

    v3 = att_edge.reshape(-1) @ W_edge
    v16 = jnp.zeros((16,), F32).at[1:4].set(v3)
    asv = att_src.reshape(1, C)
    adv = att_dst.reshape(1, C)

    pad = EPAD - E
    srcp = jnp.concatenate([src, jnp.zeros((pad,), src.dtype)]).reshape(NROWS, 128)
    dstp = jnp.concatenate([dst, jnp.full((pad,), N, dst.dtype)]).reshape(NROWS, 128)
    eap = jnp.pad(edge_attr, ((0, pad), (0, 0)))
    ea0 = eap[:, 0].reshape(NROWS, 128)
    ea1 = eap[:, 1].reshape(NROWS, 128)
    ea2 = eap[:, 2].reshape(NROWS, 128)

    xs, asrc, adst = _ka1(x, W.T, asv, adv)

    asrc_p = jnp.pad(asrc[:, 0], (0, NP - N))
    adst_p = jnp.pad(adst[:, 0], (0, NP - N))
    znp = jnp.zeros((NP,), F32)
    zrow = jnp.zeros((128, C), F32)

    e2d, den_part, deg_part, sae_part = _ksc1(
        srcp, dstp, ea0, ea1, ea2, asrc_p, adst_p, v16, znp)

    out_part = _ksc2(srcp, dstp, e2d, xs, zrow)

    denp = den_part.reshape(NTILES, NP)[:, :N].T
    degp = deg_part.reshape(NTILES, NP)[:, :N].T
    saep = sae_part.reshape(NTILES, NP)[:, :N].T
    h, s0, q0 = _k4a(out_part[0, :N, :], out_part[1, :N, :], xs,
                     denp, degp, saep, asrc, adst)
    m0 = s0 / N
    v0 = q0 / N - m0 * m0
    sc0 = bn0_gamma.reshape(1, C) / jnp.sqrt(v0 + 1e-5)
    sh0 = bn0_beta.reshape(1, C) - m0 * sc0

    z, s1, q1 = _k4b(h, sc0, sh0, lin_W.T)
    m1 = s1 / N
    v1 = q1 / N - m1 * m1
    sc1 = bn1_gamma.reshape(1, C) / jnp.sqrt(v1 + 1e-5)
    sh1 = bn1_beta.reshape(1, C) - m1 * sc1

    return _k4c(z, sc1, sh1)


# speedup vs baseline: 1.1016x; 1.0571x over previous
import jax
import jax.numpy as jnp
from jax import lax
from jax.experimental import pallas as pl
from jax.experimental.pallas import tpu as pltpu
from jax.experimental.pallas import tpu_sc as plsc

N = 10000
E = 320000
C = 128
NP = 10112
NTILES = 32
ROWS_PER_TILE = 80
NROWS = NTILES * ROWS_PER_TILE
EPAD = NROWS * 128
SLAB = NP // 16
W0_SC2 = 144
W1_SC2 = 16
F32 = jnp.float32


def _ka1_body(x_ref, wt_ref, asv_ref, adv_ref, xs_ref, asrc_ref, adst_ref):
    xb = jnp.dot(x_ref[...], wt_ref[...], preferred_element_type=F32)
    xs_ref[...] = xb
    asrc_ref[...] = jnp.sum(xb * asv_ref[...], axis=1, keepdims=True)
    adst_ref[...] = jnp.sum(xb * adv_ref[...], axis=1, keepdims=True)


def _ka1(x, wt, asv, adv):
    bn = 1000
    return pl.pallas_call(
        _ka1_body,
        grid=(N // bn,),
        in_specs=[
            pl.BlockSpec((bn, C), lambda i: (i, 0)),
            pl.BlockSpec((C, C), lambda i: (0, 0)),
            pl.BlockSpec((1, C), lambda i: (0, 0)),
            pl.BlockSpec((1, C), lambda i: (0, 0)),
        ],
        out_specs=[
            pl.BlockSpec((bn, C), lambda i: (i, 0)),
            pl.BlockSpec((bn, 1), lambda i: (i, 0)),
            pl.BlockSpec((bn, 1), lambda i: (i, 0)),
        ],
        out_shape=[
            jax.ShapeDtypeStruct((N, C), F32),
            jax.ShapeDtypeStruct((N, 1), F32),
            jax.ShapeDtypeStruct((N, 1), F32),
        ],
    )(x, wt, asv, adv)


def _ksc1_body(src2d, dst2d, ea0, ea1, ea2, asrc_h, adst_h, v16_h, znp_h,
               e2d, den_part, deg_part, sae_part,
               asrc_t, adst_t, den_t, deg_t, sae_t,
               src_sb, dst_sb, ea0_sb, ea1_sb, ea2_sb, e_sb, vbuf):
    cid = lax.axis_index("c")
    sid = lax.axis_index("s")
    wid = sid * 2 + cid

    pltpu.sync_copy(asrc_h, asrc_t)
    pltpu.sync_copy(adst_h, adst_t)
    pltpu.sync_copy(znp_h, den_t)
    pltpu.sync_copy(znp_h, deg_t)
    pltpu.sync_copy(znp_h, sae_t)
    pltpu.sync_copy(v16_h, vbuf)
    idx0 = jnp.zeros((16,), jnp.int32)
    v0b = plsc.load_gather(vbuf, [idx0 + 1])
    v1b = plsc.load_gather(vbuf, [idx0 + 2])
    v2b = plsc.load_gather(vbuf, [idx0 + 3])
    ones16 = jnp.full((16,), 1.0, F32)

    ns = 8
    for s5 in range(ROWS_PER_TILE // ns):
        r0 = wid * ROWS_PER_TILE + s5 * ns
        pltpu.sync_copy(src2d.at[pl.ds(r0, ns)], src_sb)
        pltpu.sync_copy(dst2d.at[pl.ds(r0, ns)], dst_sb)
        pltpu.sync_copy(ea0.at[pl.ds(r0, ns)], ea0_sb)
        pltpu.sync_copy(ea1.at[pl.ds(r0, ns)], ea1_sb)
        pltpu.sync_copy(ea2.at[pl.ds(r0, ns)], ea2_sb)

        def row_body(j, carry):
            for k in range(8):
                sl = pl.ds(16 * k, 16)
                sv = src_sb[j, sl]
                dv = dst_sb[j, sl]
                ae = ea0_sb[j, sl] * v0b + ea1_sb[j, sl] * v1b + ea2_sb[j, sl] * v2b
                al = (plsc.load_gather(asrc_t, [sv])
                      + plsc.load_gather(adst_t, [dv]) + ae)
                al = jnp.where(al >= 0.0, al, al * 0.2)
                ev = jnp.exp(al)
                e_sb[j, sl] = ev
                plsc.addupdate_scatter(den_t, [dv], ev)
                plsc.addupdate_scatter(deg_t, [dv], ones16)
                plsc.addupdate_scatter(sae_t, [dv], ae)
            return carry

        lax.fori_loop(0, ns, row_body, 0)
        pltpu.sync_copy(e_sb, e2d.at[pl.ds(r0, ns)])

    pltpu.sync_copy(den_t, den_part.at[pl.ds(wid * NP, NP)])
    pltpu.sync_copy(deg_t, deg_part.at[pl.ds(wid * NP, NP)])
    pltpu.sync_copy(sae_t, sae_part.at[pl.ds(wid * NP, NP)])


def _ksc1(src2d, dst2d, ea0, ea1, ea2, asrc_p, adst_p, v16, znp):
    mesh = plsc.VectorSubcoreMesh(core_axis_name="c", subcore_axis_name="s",
                                  num_cores=2, num_subcores=16)
    f = pl.kernel(
        _ksc1_body,
        out_type=(
            jax.ShapeDtypeStruct((NROWS, 128), F32),
            jax.ShapeDtypeStruct((NTILES * NP,), F32),
            jax.ShapeDtypeStruct((NTILES * NP,), F32),
            jax.ShapeDtypeStruct((NTILES * NP,), F32),
        ),
        mesh=mesh,
        compiler_params=pltpu.CompilerParams(needs_layout_passes=False),
        scratch_types=[
            pltpu.VMEM((NP,), F32),
            pltpu.VMEM((NP,), F32),
            pltpu.VMEM((NP,), F32),
            pltpu.VMEM((NP,), F32),
            pltpu.VMEM((NP,), F32),
            pltpu.VMEM((8, 128), jnp.int32),
            pltpu.VMEM((8, 128), jnp.int32),
            pltpu.VMEM((8, 128), F32),
            pltpu.VMEM((8, 128), F32),
            pltpu.VMEM((8, 128), F32),
            pltpu.VMEM((8, 128), F32),
            pltpu.VMEM((16,), F32),
        ],
    )
    return f(src2d, dst2d, ea0, ea1, ea2, asrc_p, adst_p, v16, znp)


def _ksc2_body(src2d, dst2d, e2d, xs_h, zrow_h,
               out_part,
               src_sb, dst_sb, e_sb, buf0, buf1, acc_sh, sem0, sem1):
    cid = lax.axis_index("c")
    sid = lax.axis_index("s")

    pltpu.sync_copy(zrow_h, buf0)
    slab = sid * SLAB
    for i in range(4):
        pltpu.sync_copy(buf0, acc_sh.at[pl.ds(slab + i * 128, 128)])
    pltpu.sync_copy(buf0.at[pl.ds(0, SLAB - 512)],
                    acc_sh.at[pl.ds(slab + 512, SLAB - 512)])
    plsc.subcore_barrier()

    ns = 8
    base = jnp.where(cid == 0, sid * W0_SC2, 16 * W0_SC2 + sid * W1_SC2)
    nsup0 = W0_SC2 // ns
    nsup1 = W1_SC2 // ns

    def scale_scatter(j, buf):
        @plsc.parallel_loop(0, 128, unroll=4)
        def scale_body(r):
            sb = plsc.load_gather(
                e_sb, [jnp.full((16,), j, jnp.int32),
                       jnp.full((16,), r, jnp.int32)])
            for c in range(8):
                slc = pl.ds(16 * c, 16)
                buf[r, slc] = buf[r, slc] * sb

        pltpu.sync_copy(buf, acc_sh.at[dst_sb.at[j]], add=True)

    for s5 in range(max(nsup0, nsup1)):
        if s5 >= min(nsup0, nsup1):
            other = 1 if nsup1 > nsup0 else 0
            guard = lambda: pl.when(cid == other)
        else:
            guard = None
        r0 = base + s5 * ns

        def super_chunk(r0=r0):
            pltpu.sync_copy(src2d.at[pl.ds(r0, ns)], src_sb)
            pltpu.sync_copy(dst2d.at[pl.ds(r0, ns)], dst_sb)
            pltpu.sync_copy(e2d.at[pl.ds(r0, ns)], e_sb)
            pltpu.async_copy(xs_h.at[src_sb.at[0]], buf0, sem0)

            def pair_body(t, carry):
                j0 = 2 * t
                j1 = j0 + 1
                pltpu.make_async_copy(xs_h.at[src_sb.at[j0]], buf0,
                                      sem0).wait()
                pltpu.async_copy(xs_h.at[src_sb.at[j1]], buf1, sem1)
                scale_scatter(j0, buf0)
                pltpu.make_async_copy(xs_h.at[src_sb.at[j1]], buf1,
                                      sem1).wait()

                @pl.when(t < ns // 2 - 1)
                def _():
                    pltpu.async_copy(xs_h.at[src_sb.at[j0 + 2]], buf0, sem0)
                scale_scatter(j1, buf1)
                return carry

            lax.fori_loop(0, ns // 2, pair_body, 0)

        if guard is None:
            super_chunk()
        else:
            guard()(super_chunk)

    plsc.subcore_barrier()
    pltpu.sync_copy(acc_sh.at[pl.ds(slab, SLAB)],
                    out_part.at[cid, pl.ds(slab, SLAB)])


def _ksc2(src2d, dst2d, e2d, xs, zrow):
    mesh = plsc.VectorSubcoreMesh(core_axis_name="c", subcore_axis_name="s",
                                  num_cores=2, num_subcores=16)
    f = pl.kernel(
        _ksc2_body,
        out_type=jax.ShapeDtypeStruct((2, NP, C), F32),
        mesh=mesh,
        compiler_params=pltpu.CompilerParams(needs_layout_passes=False),
        scratch_types=[
            pltpu.VMEM((8, 128), jnp.int32),
            pltpu.VMEM((8, 128), jnp.int32),
            pltpu.VMEM((8, 128), F32),
            pltpu.VMEM((128, C), F32),
            pltpu.VMEM((128, C), F32),
            pltpu.VMEM_SHARED((NP, C), F32),
            pltpu.SemaphoreType.DMA,
            pltpu.SemaphoreType.DMA,
        ],
    )
    return f(src2d, dst2d, e2d, xs, zrow)


def _k4a_body(op0_ref, op1_ref, xs_ref, denp_ref, degp_ref, saep_ref,
              as_ref, ad_ref, h_ref, s_ref, q_ref):
    den = jnp.sum(denp_ref[...], axis=1)
    deg = jnp.sum(degp_ref[...], axis=1)
    sae = jnp.sum(saep_ref[...], axis=1)
    al = as_ref[...][:, 0] + ad_ref[...][:, 0] + sae / jnp.maximum(deg, 1.0)
    al = jnp.where(al >= 0.0, al, al * 0.2)
    el = jnp.exp(al)
    rden = 1.0 / (den + el + 1e-16)
    h = (op0_ref[...] + op1_ref[...] + el[:, None] * xs_ref[...]) * rden[:, None]
    h_ref[...] = h

    @pl.when(pl.program_id(0) == 0)
    def _():
        s_ref[...] = jnp.zeros_like(s_ref)
        q_ref[...] = jnp.zeros_like(q_ref)

    s_ref[...] += jnp.sum(h, axis=0, keepdims=True)
    q_ref[...] += jnp.sum(h * h, axis=0, keepdims=True)


def _k4a(op0, op1, xs, denp, degp, saep, asrc, adst):
    bn = 1000
    return pl.pallas_call(
        _k4a_body,
        grid=(N // bn,),
        in_specs=[
            pl.BlockSpec((bn, C), lambda i: (i, 0)),
            pl.BlockSpec((bn, C), lambda i: (i, 0)),
            pl.BlockSpec((bn, C), lambda i: (i, 0)),
            pl.BlockSpec((bn, NTILES), lambda i: (i, 0)),
            pl.BlockSpec((bn, NTILES), lambda i: (i, 0)),
            pl.BlockSpec((bn, NTILES), lambda i: (i, 0)),
            pl.BlockSpec((bn, 1), lambda i: (i, 0)),
            pl.BlockSpec((bn, 1), lambda i: (i, 0)),
        ],
        out_specs=[
            pl.BlockSpec((bn, C), lambda i: (i, 0)),
            pl.BlockSpec((1, C), lambda i: (0, 0)),
            pl.BlockSpec((1, C), lambda i: (0, 0)),
        ],
        out_shape=[
            jax.ShapeDtypeStruct((N, C), F32),
            jax.ShapeDtypeStruct((1, C), F32),
            jax.ShapeDtypeStruct((1, C), F32),
        ],
    )(op0, op1, xs, denp, degp, saep, asrc, adst)


def _k4b_body(h_ref, sc0_ref, sh0_ref, lwt_ref, z_ref, s_ref, q_ref):
    y = jnp.maximum(h_ref[...] * sc0_ref[...] + sh0_ref[...], 0.0)
    z = jnp.dot(y, lwt_ref[...], preferred_element_type=F32)
    z_ref[...] = z

    @pl.when(pl.program_id(0) == 0)
    def _():
        s_ref[...] = jnp.zeros_like(s_ref)
        q_ref[...] = jnp.zeros_like(q_ref)

    s_ref[...] += jnp.sum(z, axis=0, keepdims=True)
    q_ref[...] += jnp.sum(z * z, axis=0, keepdims=True)


def _k4b(h, sc0, sh0, lwt):
    bn = 1000
    return pl.pallas_call(
        _k4b_body,
        grid=(N // bn,),
        in_specs=[
            pl.BlockSpec((bn, C), lambda i: (i, 0)),
            pl.BlockSpec((1, C), lambda i: (0, 0)),
            pl.BlockSpec((1, C), lambda i: (0, 0)),
            pl.BlockSpec((C, C), lambda i: (0, 0)),
        ],
        out_specs=[
            pl.BlockSpec((bn, C), lambda i: (i, 0)),
            pl.BlockSpec((1, C), lambda i: (0, 0)),
            pl.BlockSpec((1, C), lambda i: (0, 0)),
        ],
        out_shape=[
            jax.ShapeDtypeStruct((N, C), F32),
            jax.ShapeDtypeStruct((1, C), F32),
            jax.ShapeDtypeStruct((1, C), F32),
        ],
    )(h, sc0, sh0, lwt)


def _k4c_body(z_ref, sc1_ref, sh1_ref, o_ref):
    o_ref[...] = jnp.maximum(z_ref[...] * sc1_ref[...] + sh1_ref[...], 0.0)


def _k4c(z, sc1, sh1):
    bn = 1000
    return pl.pallas_call(
        _k4c_body,
        grid=(N // bn,),
        in_specs=[
            pl.BlockSpec((bn, C), lambda i: (i, 0)),
            pl.BlockSpec((1, C), lambda i: (0, 0)),
            pl.BlockSpec((1, C), lambda i: (0, 0)),
        ],
        out_specs=pl.BlockSpec((bn, C), lambda i: (i, 0)),
        out_shape=jax.ShapeDtypeStruct((N, C), F32),
    )(z, sc1, sh1)


def kernel(x, adj, edge_attr, W, att_src, att_dst, W_edge, att_edge, bias,
           bn0_gamma, bn0_beta, lin_W, lin_b, bn1_gamma, bn1_beta):
    del bias, lin_b
    src, dst = adj[0], adj[1]

    v3 = att_edge.reshape(-1) @ W_edge
    v16 = jnp.zeros((16,), F32).at[1:4].set(v3)
    asv = att_src.reshape(1, C)
    adv = att_dst.reshape(1, C)

    pad = EPAD - E
    srcp = jnp.concatenate([src, jnp.zeros((pad,), src.dtype)]).reshape(NROWS, 128)
    dstp = jnp.concatenate([dst, jnp.full((pad,), N, dst.dtype)]).reshape(NROWS, 128)
    eap = jnp.pad(edge_attr, ((0, pad), (0, 0)))
    ea0 = eap[:, 0].reshape(NROWS, 128)
    ea1 = eap[:, 1].reshape(NROWS, 128)
    ea2 = eap[:, 2].reshape(NROWS, 128)

    xs, asrc, adst = _ka1(x, W.T, asv, adv)

    asrc_p = jnp.pad(asrc[:, 0], (0, NP - N))
    adst_p = jnp.pad(adst[:, 0], (0, NP - N))
    znp = jnp.zeros((NP,), F32)
    zrow = jnp.zeros((128, C), F32)

    e2d, den_part, deg_part, sae_part = _ksc1(
        srcp, dstp, ea0, ea1, ea2, asrc_p, adst_p, v16, znp)

    out_part = _ksc2(srcp, dstp, e2d, xs, zrow)

    denp = den_part.reshape(NTILES, NP)[:, :N].T
    degp = deg_part.reshape(NTILES, NP)[:, :N].T
    saep = sae_part.reshape(NTILES, NP)[:, :N].T
    h, s0, q0 = _k4a(out_part[0, :N, :], out_part[1, :N, :], xs,
                     denp, degp, saep, asrc, adst)
    m0 = s0 / N
    v0 = q0 / N - m0 * m0
    sc0 = bn0_gamma.reshape(1, C) / jnp.sqrt(v0 + 1e-5)
    sh0 = bn0_beta.reshape(1, C) - m0 * sc0

    z, s1, q1 = _k4b(h, sc0, sh0, lin_W.T)
    m1 = s1 / N
    v1 = q1 / N - m1 * m1
    sc1 = bn1_gamma.reshape(1, C) / jnp.sqrt(v1 + 1e-5)
    sh1 = bn1_beta.reshape(1, C) - m1 * sc1

    return _k4c(z, sc1, sh1)
